# Initial kernel scaffold; baseline (speedup 1.0000x reference)
import functools

import jax
import jax.numpy as jnp
from jax import lax
from jax.experimental import pallas as pl
from jax.experimental.pallas import tpu as pltpu
from jax.experimental.pallas import tpu_sc as plsc

N = 10000
F = 256
CLS = 3
CP = 16
NC, NS = 2, 16
NPAD = 10240
E = 160000
EPAD = 163840
B = 128
FH = F // NC
STRIPE = NPAD // NS
RB = 512

_mesh = plsc.VectorSubcoreMesh(core_axis_name="c", subcore_axis_name="s")


@functools.partial(
    pl.kernel,
    mesh=_mesh,
    out_type=jax.ShapeDtypeStruct((NC, NPAD, CP), jnp.float32),
    scratch_types=[
        pltpu.VMEM((B,), jnp.int32),
        pltpu.VMEM((B, CP), jnp.float32),
        pltpu.VMEM_SHARED((NPAD, CP), jnp.float32),
    ],
)
def _deg_kernel(dst_hbm, zeros_hbm, out_hbm, dst_v, ones_v, accum):
    c = lax.axis_index("c")
    s = lax.axis_index("s")

    @pl.loop(0, B)
    def _(i):
        ones_v[i, :] = jnp.ones((CP,), jnp.float32)

    pltpu.sync_copy(zeros_hbm, accum.at[pl.ds(s * STRIPE, STRIPE)])
    plsc.subcore_barrier()

    half = EPAD // NC
    base = c * half + s * (half // NS)

    @pl.loop(0, half // NS // B)
    def _(k):
        pltpu.sync_copy(dst_hbm.at[pl.ds(base + k * B, B)], dst_v)
        pltpu.sync_copy(ones_v, accum.at[dst_v], add=True)

    plsc.subcore_barrier()
    pltpu.sync_copy(
        accum.at[pl.ds(s * STRIPE, STRIPE)],
        out_hbm.at[c].at[pl.ds(s * STRIPE, STRIPE)],
    )


@functools.partial(
    pl.kernel,
    mesh=_mesh,
    out_type=jax.ShapeDtypeStruct((NC, NPAD, FH), jnp.float32),
    scratch_types=[
        pltpu.VMEM((B,), jnp.int32),
        pltpu.VMEM((B,), jnp.int32),
        pltpu.VMEM((B, FH), jnp.float32),
        pltpu.VMEM_SHARED((NPAD, FH), jnp.float32),
        pltpu.SemaphoreType.DMA,
    ],
)
def _agg1_kernel(hp_hbm, src_hbm, dst_hbm, zeros_hbm, out_hbm,
                 src_v, dst_v, rows_v, accum, sem):
    c = lax.axis_index("c")
    s = lax.axis_index("s")

    pltpu.sync_copy(zeros_hbm, accum.at[pl.ds(s * STRIPE, STRIPE)])
    plsc.subcore_barrier()

    base = s * (EPAD // NS)

    @pl.loop(0, EPAD // NS // B)
    def _(k):
        off = base + k * B
        pltpu.sync_copy(src_hbm.at[pl.ds(off, B)], src_v)
        pltpu.sync_copy(dst_hbm.at[pl.ds(off, B)], dst_v)
        pltpu.async_copy(hp_hbm.at[c].at[src_v], rows_v, sem).wait()
        pltpu.sync_copy(rows_v, accum.at[dst_v], add=True)

    plsc.subcore_barrier()
    pltpu.sync_copy(
        accum.at[pl.ds(s * STRIPE, STRIPE)],
        out_hbm.at[c].at[pl.ds(s * STRIPE, STRIPE)],
    )


@functools.partial(
    pl.kernel,
    mesh=_mesh,
    out_type=jax.ShapeDtypeStruct((NC, NPAD, CP), jnp.float32),
    scratch_types=[
        pltpu.VMEM((B,), jnp.int32),
        pltpu.VMEM((B,), jnp.int32),
        pltpu.VMEM((B, CP), jnp.float32),
        pltpu.VMEM_SHARED((NPAD, CP), jnp.float32),
        pltpu.SemaphoreType.DMA,
    ],
)
def _agg2_kernel(hp_hbm, src_hbm, dst_hbm, zeros_hbm, out_hbm,
                 src_v, dst_v, rows_v, accum, sem):
    c = lax.axis_index("c")
    s = lax.axis_index("s")

    pltpu.sync_copy(zeros_hbm, accum.at[pl.ds(s * STRIPE, STRIPE)])
    plsc.subcore_barrier()

    half = EPAD // NC
    base = c * half + s * (half // NS)

    @pl.loop(0, half // NS // B)
    def _(k):
        off = base + k * B
        pltpu.sync_copy(src_hbm.at[pl.ds(off, B)], src_v)
        pltpu.sync_copy(dst_hbm.at[pl.ds(off, B)], dst_v)
        pltpu.async_copy(hp_hbm.at[src_v], rows_v, sem).wait()
        pltpu.sync_copy(rows_v, accum.at[dst_v], add=True)

    plsc.subcore_barrier()
    pltpu.sync_copy(
        accum.at[pl.ds(s * STRIPE, STRIPE)],
        out_hbm.at[c].at[pl.ds(s * STRIPE, STRIPE)],
    )


def _mm1_body(x_ref, w_ref, o_ref):
    o_ref[...] = jnp.dot(x_ref[...], w_ref[...],
                         preferred_element_type=jnp.float32,
                         precision=lax.Precision.HIGHEST)


def _dis_of(deg_ref):
    deg = deg_ref[0, :, 0:1] + deg_ref[1, :, 0:1] + 1.0
    return lax.rsqrt(deg)


def _scale_body(deg_ref, h_ref, o_ref):
    o_ref[0] = _dis_of(deg_ref) * h_ref[...]


def _layer2_body(deg_ref, a_ref, hp_ref, b1_ref, w2_ref, o_ref):
    dis = _dis_of(deg_ref)
    agg = jnp.concatenate([a_ref[0] + hp_ref[0], a_ref[1] + hp_ref[1]], axis=1)
    h1 = jnp.maximum(dis * agg + b1_ref[...], 0.0)
    o_ref[...] = dis * jnp.dot(h1, w2_ref[...],
                               preferred_element_type=jnp.float32,
                               precision=lax.Precision.HIGHEST)


def _final_body(deg_ref, a_ref, hp2_ref, b2_ref, o_ref):
    dis = _dis_of(deg_ref)
    o_ref[...] = dis * (a_ref[0] + a_ref[1] + hp2_ref[...]) + b2_ref[...]


def kernel(x, edge_index, W1, b1, W2, b2):
    f32 = jnp.float32
    src = edge_index[0].astype(jnp.int32)
    dst = edge_index[1].astype(jnp.int32)
    pad = jnp.full((EPAD - E,), N, jnp.int32)
    src = jnp.concatenate([src, pad])
    dst = jnp.concatenate([dst, pad])

    xp = jnp.pad(x, ((0, NPAD - N), (0, 0)))
    b1r = b1.reshape(1, F)
    w2p = jnp.pad(W2, ((0, 0), (0, CP - CLS)))
    b2r = jnp.pad(b2, (0, CP - CLS)).reshape(1, CP)

    zeros_cp = jnp.zeros((STRIPE, CP), f32)
    zeros_fh = jnp.zeros((STRIPE, FH), f32)

    deg = _deg_kernel(dst, zeros_cp)

    grid = NPAD // RB
    hraw = pl.pallas_call(
        _mm1_body,
        grid=(grid,),
        in_specs=[pl.BlockSpec((RB, F), lambda i: (i, 0)),
                  pl.BlockSpec((F, F), lambda i: (0, 0))],
        out_specs=pl.BlockSpec((RB, F), lambda i: (i, 0)),
        out_shape=jax.ShapeDtypeStruct((NPAD, F), f32),
    )(xp, W1)

    hp = pl.pallas_call(
        _scale_body,
        grid=(NC, grid),
        in_specs=[pl.BlockSpec((NC, RB, CP), lambda c, i: (0, i, 0)),
                  pl.BlockSpec((RB, FH), lambda c, i: (i, c))],
        out_specs=pl.BlockSpec((1, RB, FH), lambda c, i: (c, i, 0)),
        out_shape=jax.ShapeDtypeStruct((NC, NPAD, FH), f32),
    )(deg, hraw)

    agg1 = _agg1_kernel(hp, src, dst, zeros_fh)

    hp2 = pl.pallas_call(
        _layer2_body,
        grid=(grid,),
        in_specs=[pl.BlockSpec((NC, RB, CP), lambda i: (0, i, 0)),
                  pl.BlockSpec((NC, RB, FH), lambda i: (0, i, 0)),
                  pl.BlockSpec((NC, RB, FH), lambda i: (0, i, 0)),
                  pl.BlockSpec((1, F), lambda i: (0, 0)),
                  pl.BlockSpec((F, CP), lambda i: (0, 0))],
        out_specs=pl.BlockSpec((RB, CP), lambda i: (i, 0)),
        out_shape=jax.ShapeDtypeStruct((NPAD, CP), f32),
    )(deg, agg1, hp, b1r, w2p)

    agg2 = _agg2_kernel(hp2, src, dst, zeros_cp)

    out = pl.pallas_call(
        _final_body,
        grid=(grid,),
        in_specs=[pl.BlockSpec((NC, RB, CP), lambda i: (0, i, 0)),
                  pl.BlockSpec((NC, RB, CP), lambda i: (0, i, 0)),
                  pl.BlockSpec((RB, CP), lambda i: (i, 0)),
                  pl.BlockSpec((1, CP), lambda i: (0, 0))],
        out_specs=pl.BlockSpec((RB, CP), lambda i: (i, 0)),
        out_shape=jax.ShapeDtypeStruct((NPAD, CP), f32),
    )(deg, agg2, hp2, b2r)

    return out[:N, :CLS]


# speedup vs baseline: 5.9717x; 5.9717x over previous
import dataclasses
import functools

import jax
import jax.numpy as jnp
from jax import lax
from jax.experimental import pallas as pl
from jax.experimental.pallas import tpu as pltpu
from jax.experimental.pallas import tpu_sc as plsc

N = 10000
F = 256
CLS = 3
CP2 = 128
NC, NS = 2, 16
NW = NC * NS
L = 16
NPAD = 10240
E = 160000
EPAD = 163840
B = 128
FH = F // NC
STRIPE = NPAD // NS
RB = 512

_mesh = plsc.VectorSubcoreMesh(core_axis_name="c", subcore_axis_name="s")

_cp_no_layout = pltpu.CompilerParams()
if "needs_layout_passes" in pltpu.CompilerParams.__dataclass_fields__:
    _cp_no_layout = dataclasses.replace(_cp_no_layout, needs_layout_passes=False)


@functools.partial(
    pl.kernel,
    mesh=_mesh,
    out_type=jax.ShapeDtypeStruct((NC, NS, NPAD), jnp.float32),
    scratch_types=[
        pltpu.VMEM((B,), jnp.int32),
        pltpu.VMEM((NPAD,), jnp.float32),
    ],
    compiler_params=_cp_no_layout,
)
def _deg_kernel(dst_hbm, zeros_hbm, out_hbm, dst_v, hist):
    c = lax.axis_index("c")
    s = lax.axis_index("s")
    pltpu.sync_copy(zeros_hbm, hist)
    ones16 = jnp.ones((L,), jnp.float32)
    per_w = EPAD // NW
    base = (c * NS + s) * per_w

    @pl.loop(0, per_w // B)
    def _(k):
        pltpu.sync_copy(dst_hbm.at[pl.ds(base + k * B, B)], dst_v)

        @pl.loop(0, B, step=L)
        def _(j):
            plsc.addupdate_scatter(hist, [dst_v[pl.ds(j, L)]], ones16)

    pltpu.sync_copy(hist, out_hbm.at[c].at[s])


@functools.partial(
    pl.kernel,
    mesh=_mesh,
    out_type=jax.ShapeDtypeStruct((NC, NPAD, FH), jnp.float32),
    scratch_types=[
        pltpu.VMEM((B,), jnp.int32),
        pltpu.VMEM((B,), jnp.int32),
        pltpu.VMEM((B, FH), jnp.float32),
        pltpu.VMEM_SHARED((NPAD, FH), jnp.float32),
        pltpu.SemaphoreType.DMA,
    ],
)
def _agg1_kernel(hp_hbm, src_hbm, dst_hbm, zeros_hbm, out_hbm,
                 src_v, dst_v, rows_v, accum, sem):
    c = lax.axis_index("c")
    s = lax.axis_index("s")

    pltpu.sync_copy(zeros_hbm, accum.at[pl.ds(s * STRIPE, STRIPE)])
    plsc.subcore_barrier()

    base = s * (EPAD // NS)

    @pl.loop(0, EPAD // NS // B)
    def _(k):
        off = base + k * B
        pltpu.sync_copy(src_hbm.at[pl.ds(off, B)], src_v)
        pltpu.sync_copy(dst_hbm.at[pl.ds(off, B)], dst_v)
        pltpu.async_copy(hp_hbm.at[c].at[src_v], rows_v, sem).wait()
        pltpu.sync_copy(rows_v, accum.at[dst_v], add=True)

    plsc.subcore_barrier()
    pltpu.sync_copy(
        accum.at[pl.ds(s * STRIPE, STRIPE)],
        out_hbm.at[c].at[pl.ds(s * STRIPE, STRIPE)],
    )


@functools.partial(
    pl.kernel,
    mesh=_mesh,
    out_type=jax.ShapeDtypeStruct((NC, NPAD, CP2), jnp.float32),
    scratch_types=[
        pltpu.VMEM((B,), jnp.int32),
        pltpu.VMEM((B,), jnp.int32),
        pltpu.VMEM((B, CP2), jnp.float32),
        pltpu.VMEM_SHARED((NPAD, CP2), jnp.float32),
        pltpu.SemaphoreType.DMA,
    ],
)
def _agg2_kernel(hp_hbm, src_hbm, dst_hbm, zeros_hbm, out_hbm,
                 src_v, dst_v, rows_v, accum, sem):
    c = lax.axis_index("c")
    s = lax.axis_index("s")

    pltpu.sync_copy(zeros_hbm, accum.at[pl.ds(s * STRIPE, STRIPE)])
    plsc.subcore_barrier()

    half = EPAD // NC
    base = c * half + s * (half // NS)

    @pl.loop(0, half // NS // B)
    def _(k):
        off = base + k * B
        pltpu.sync_copy(src_hbm.at[pl.ds(off, B)], src_v)
        pltpu.sync_copy(dst_hbm.at[pl.ds(off, B)], dst_v)
        pltpu.async_copy(hp_hbm.at[src_v], rows_v, sem).wait()
        pltpu.sync_copy(rows_v, accum.at[dst_v], add=True)

    plsc.subcore_barrier()
    pltpu.sync_copy(
        accum.at[pl.ds(s * STRIPE, STRIPE)],
        out_hbm.at[c].at[pl.ds(s * STRIPE, STRIPE)],
    )


def _mm1_body(x_ref, w_ref, o_ref):
    o_ref[...] = jnp.dot(x_ref[...], w_ref[...],
                         preferred_element_type=jnp.float32,
                         precision=lax.Precision.HIGHEST)


def _dis_of(deg_ref):
    deg = jnp.sum(deg_ref[...], axis=(0, 1)) + 1.0
    return lax.rsqrt(deg)[:, None]


def _scale_body(deg_ref, h_ref, o_ref):
    o_ref[0] = _dis_of(deg_ref) * h_ref[...]


def _layer2_body(deg_ref, a_ref, hp_ref, b1_ref, w2_ref, o_ref):
    dis = _dis_of(deg_ref)
    agg = jnp.concatenate([a_ref[0] + hp_ref[0], a_ref[1] + hp_ref[1]], axis=1)
    h1 = jnp.maximum(dis * agg + b1_ref[...], 0.0)
    o_ref[...] = dis * jnp.dot(h1, w2_ref[...],
                               preferred_element_type=jnp.float32,
                               precision=lax.Precision.HIGHEST)


def _final_body(deg_ref, a_ref, hp2_ref, b2_ref, o_ref):
    dis = _dis_of(deg_ref)
    o_ref[...] = dis * (a_ref[0] + a_ref[1] + hp2_ref[...]) + b2_ref[...]


def kernel(x, edge_index, W1, b1, W2, b2):
    f32 = jnp.float32
    src = edge_index[0].astype(jnp.int32)
    dst = edge_index[1].astype(jnp.int32)
    pad = jnp.full((EPAD - E,), N, jnp.int32)
    src = jnp.concatenate([src, pad])
    dst = jnp.concatenate([dst, pad])

    xp = jnp.pad(x, ((0, NPAD - N), (0, 0)))
    b1r = b1.reshape(1, F)
    w2p = jnp.pad(W2, ((0, 0), (0, CP2 - CLS)))
    b2r = jnp.pad(b2, (0, CP2 - CLS)).reshape(1, CP2)

    zeros_n = jnp.zeros((NPAD,), f32)
    zeros_fh = jnp.zeros((STRIPE, FH), f32)
    zeros_cp2 = jnp.zeros((STRIPE, CP2), f32)

    deg = _deg_kernel(dst, zeros_n)

    grid = NPAD // RB
    hraw = pl.pallas_call(
        _mm1_body,
        grid=(grid,),
        in_specs=[pl.BlockSpec((RB, F), lambda i: (i, 0)),
                  pl.BlockSpec((F, F), lambda i: (0, 0))],
        out_specs=pl.BlockSpec((RB, F), lambda i: (i, 0)),
        out_shape=jax.ShapeDtypeStruct((NPAD, F), f32),
    )(xp, W1)

    hp = pl.pallas_call(
        _scale_body,
        grid=(NC, grid),
        in_specs=[pl.BlockSpec((NC, NS, RB), lambda c, i: (0, 0, i)),
                  pl.BlockSpec((RB, FH), lambda c, i: (i, c))],
        out_specs=pl.BlockSpec((1, RB, FH), lambda c, i: (c, i, 0)),
        out_shape=jax.ShapeDtypeStruct((NC, NPAD, FH), f32),
    )(deg, hraw)

    agg1 = _agg1_kernel(hp, src, dst, zeros_fh)

    hp2 = pl.pallas_call(
        _layer2_body,
        grid=(grid,),
        in_specs=[pl.BlockSpec((NC, NS, RB), lambda i: (0, 0, i)),
                  pl.BlockSpec((NC, RB, FH), lambda i: (0, i, 0)),
                  pl.BlockSpec((NC, RB, FH), lambda i: (0, i, 0)),
                  pl.BlockSpec((1, F), lambda i: (0, 0)),
                  pl.BlockSpec((F, CP2), lambda i: (0, 0))],
        out_specs=pl.BlockSpec((RB, CP2), lambda i: (i, 0)),
        out_shape=jax.ShapeDtypeStruct((NPAD, CP2), f32),
    )(deg, agg1, hp, b1r, w2p)

    agg2 = _agg2_kernel(hp2, src, dst, zeros_cp2)

    out = pl.pallas_call(
        _final_body,
        grid=(grid,),
        in_specs=[pl.BlockSpec((NC, NS, RB), lambda i: (0, 0, i)),
                  pl.BlockSpec((NC, RB, CP2), lambda i: (0, i, 0)),
                  pl.BlockSpec((RB, CP2), lambda i: (i, 0)),
                  pl.BlockSpec((1, CP2), lambda i: (0, 0))],
        out_specs=pl.BlockSpec((RB, CP2), lambda i: (i, 0)),
        out_shape=jax.ShapeDtypeStruct((NPAD, CP2), f32),
    )(deg, agg2, hp2, b2r)

    return out[:N, :CLS]


# speedup vs baseline: 7.6544x; 1.2818x over previous
import dataclasses
import functools

import jax
import jax.numpy as jnp
from jax import lax
from jax.experimental import pallas as pl
from jax.experimental.pallas import tpu as pltpu
from jax.experimental.pallas import tpu_sc as plsc

N = 10000
F = 256
CLS = 3
CP2 = 128
NC, NS = 2, 16
NW = NC * NS
L = 16
NPAD = 10240
E = 160000
EPAD = 163840
B = 128
FH = F // NC
STRIPE = NPAD // NS
RB = 512

_mesh = plsc.VectorSubcoreMesh(core_axis_name="c", subcore_axis_name="s")

_cp_no_layout = pltpu.CompilerParams()
if "needs_layout_passes" in pltpu.CompilerParams.__dataclass_fields__:
    _cp_no_layout = dataclasses.replace(_cp_no_layout, needs_layout_passes=False)


@functools.partial(
    pl.kernel,
    mesh=_mesh,
    out_type=jax.ShapeDtypeStruct((NC, NS, NPAD), jnp.float32),
    scratch_types=[
        pltpu.VMEM((B,), jnp.int32),
        pltpu.VMEM((NPAD,), jnp.float32),
    ],
    compiler_params=_cp_no_layout,
)
def _deg_kernel(dst_hbm, zeros_hbm, out_hbm, dst_v, hist):
    c = lax.axis_index("c")
    s = lax.axis_index("s")
    pltpu.sync_copy(zeros_hbm, hist)
    ones16 = jnp.ones((L,), jnp.float32)
    per_w = EPAD // NW
    base = (c * NS + s) * per_w

    @pl.loop(0, per_w // B)
    def _(k):
        pltpu.sync_copy(dst_hbm.at[pl.ds(base + k * B, B)], dst_v)

        @pl.loop(0, B, step=L)
        def _(j):
            plsc.addupdate_scatter(hist, [dst_v[pl.ds(j, L)]], ones16)

    pltpu.sync_copy(hist, out_hbm.at[c].at[s])


NROWS = 2
NIDX = 4


def _make_agg(feat_w, split_features):
    per_tile = EPAD // NS if split_features else EPAD // NC // NS
    K = per_tile // B

    @functools.partial(
        pl.kernel,
        mesh=_mesh,
        out_type=jax.ShapeDtypeStruct((NC, NPAD, feat_w), jnp.float32),
        scratch_types=(
            [pltpu.VMEM((K, B), jnp.int32)]
            + [pltpu.VMEM((B,), jnp.int32) for _ in range(NIDX)]
            + [pltpu.VMEM((B, feat_w), jnp.float32) for _ in range(NROWS)]
            + [pltpu.VMEM_SHARED((NPAD, feat_w), jnp.float32)]
            + [pltpu.SemaphoreType.DMA for _ in range(NROWS + NIDX)]
        ),
    )
    def agg(hp_hbm, srcr_hbm, dstr_hbm, zeros_hbm, out_hbm, dst_all, *rest):
        isrc = rest[:NIDX]
        rows = rest[NIDX:NIDX + NROWS]
        accum = rest[NIDX + NROWS]
        gsem = rest[NIDX + NROWS + 1:NIDX + NROWS + 1 + NROWS]
        isem = rest[NIDX + NROWS + 1 + NROWS:]
        c = lax.axis_index("c")
        s = lax.axis_index("s")
        hp = hp_hbm.at[c] if split_features else hp_hbm
        w = s if split_features else c * NS + s
        srcw = srcr_hbm.at[w]

        def load_idx(k, jj):
            pltpu.async_copy(srcw.at[k], isrc[jj], isem[jj])

        def wait_idx(jj):
            pltpu.make_async_copy(srcw.at[0], isrc[jj], isem[jj]).wait()

        def start_gather(jj, j):
            pltpu.async_copy(hp.at[isrc[jj]], rows[j], gsem[j])

        def wait_gather(j):
            pltpu.make_async_copy(hp.at[isrc[0]], rows[j], gsem[j]).wait()

        def scatter(j, k):
            pltpu.sync_copy(rows[j], accum.at[dst_all.at[k]], add=True)

        pltpu.sync_copy(zeros_hbm, accum.at[pl.ds(s * STRIPE, STRIPE)])
        pltpu.sync_copy(dstr_hbm.at[w], dst_all)
        for t in range(NIDX):
            load_idx(t, t)
        wait_idx(0)
        start_gather(0, 0)
        wait_idx(1)
        start_gather(1, 1)
        plsc.subcore_barrier()

        @pl.loop(0, K // NIDX - 1)
        def _(q):
            for t in range(NIDX):
                k = q * NIDX + t
                j = t % NROWS
                wait_gather(j)
                scatter(j, k)
                load_idx(k + NIDX, t)
                wait_idx((t + NROWS) % NIDX)
                start_gather((t + NROWS) % NIDX, j)

        for t in range(NIDX):
            k = K - NIDX + t
            j = t % NROWS
            wait_gather(j)
            scatter(j, k)
            if t < NIDX - NROWS:
                wait_idx((t + NROWS) % NIDX)
                start_gather((t + NROWS) % NIDX, j)

        plsc.subcore_barrier()
        pltpu.sync_copy(
            accum.at[pl.ds(s * STRIPE, STRIPE)],
            out_hbm.at[c].at[pl.ds(s * STRIPE, STRIPE)],
        )

    return agg


_agg1_kernel = _make_agg(FH, True)
_agg2_kernel = _make_agg(CP2, False)


def _mm1_body(x_ref, w_ref, o_ref):
    o_ref[...] = jnp.dot(x_ref[...], w_ref[...],
                         preferred_element_type=jnp.float32,
                         precision=lax.Precision.HIGHEST)


def _dis_of(deg_ref):
    deg = jnp.sum(deg_ref[...], axis=(0, 1)) + 1.0
    return lax.rsqrt(deg)[:, None]


def _scale_body(deg_ref, h_ref, o_ref):
    o_ref[0] = _dis_of(deg_ref) * h_ref[...]


def _layer2_body(deg_ref, a_ref, hp_ref, b1_ref, w2_ref, o_ref):
    dis = _dis_of(deg_ref)
    agg = jnp.concatenate([a_ref[0] + hp_ref[0], a_ref[1] + hp_ref[1]], axis=1)
    h1 = jnp.maximum(dis * agg + b1_ref[...], 0.0)
    o_ref[...] = dis * jnp.dot(h1, w2_ref[...],
                               preferred_element_type=jnp.float32,
                               precision=lax.Precision.HIGHEST)


def _final_body(deg_ref, a_ref, hp2_ref, b2_ref, o_ref):
    dis = _dis_of(deg_ref)
    o_ref[...] = dis * (a_ref[0] + a_ref[1] + hp2_ref[...]) + b2_ref[...]


def kernel(x, edge_index, W1, b1, W2, b2):
    f32 = jnp.float32
    src = edge_index[0].astype(jnp.int32)
    dst = edge_index[1].astype(jnp.int32)
    pad = jnp.full((EPAD - E,), N, jnp.int32)
    src = jnp.concatenate([src, pad])
    dst = jnp.concatenate([dst, pad])

    xp = jnp.pad(x, ((0, NPAD - N), (0, 0)))
    b1r = b1.reshape(1, F)
    w2p = jnp.pad(W2, ((0, 0), (0, CP2 - CLS)))
    b2r = jnp.pad(b2, (0, CP2 - CLS)).reshape(1, CP2)

    zeros_n = jnp.zeros((NPAD,), f32)
    zeros_fh = jnp.zeros((STRIPE, FH), f32)

    k1 = EPAD // NS // B
    k2 = EPAD // NC // NS // B
    src_r1 = src.reshape(NS, k1, B)
    dst_r1 = dst.reshape(NS, k1, B)
    src_r2 = src.reshape(NC * NS, k2, B)
    dst_r2 = dst.reshape(NC * NS, k2, B)

    deg = _deg_kernel(dst, zeros_n)

    grid = NPAD // RB
    hraw = pl.pallas_call(
        _mm1_body,
        grid=(grid,),
        in_specs=[pl.BlockSpec((RB, F), lambda i: (i, 0)),
                  pl.BlockSpec((F, F), lambda i: (0, 0))],
        out_specs=pl.BlockSpec((RB, F), lambda i: (i, 0)),
        out_shape=jax.ShapeDtypeStruct((NPAD, F), f32),
    )(xp, W1)

    hp = pl.pallas_call(
        _scale_body,
        grid=(NC, grid),
        in_specs=[pl.BlockSpec((NC, NS, RB), lambda c, i: (0, 0, i)),
                  pl.BlockSpec((RB, FH), lambda c, i: (i, c))],
        out_specs=pl.BlockSpec((1, RB, FH), lambda c, i: (c, i, 0)),
        out_shape=jax.ShapeDtypeStruct((NC, NPAD, FH), f32),
    )(deg, hraw)

    agg1 = _agg1_kernel(hp, src_r1, dst_r1, zeros_fh)

    hp2 = pl.pallas_call(
        _layer2_body,
        grid=(grid,),
        in_specs=[pl.BlockSpec((NC, NS, RB), lambda i: (0, 0, i)),
                  pl.BlockSpec((NC, RB, FH), lambda i: (0, i, 0)),
                  pl.BlockSpec((NC, RB, FH), lambda i: (0, i, 0)),
                  pl.BlockSpec((1, F), lambda i: (0, 0)),
                  pl.BlockSpec((F, CP2), lambda i: (0, 0))],
        out_specs=pl.BlockSpec((RB, CP2), lambda i: (i, 0)),
        out_shape=jax.ShapeDtypeStruct((NPAD, CP2), f32),
    )(deg, agg1, hp, b1r, w2p)

    agg2 = _agg2_kernel(hp2, src_r2, dst_r2, zeros_fh)

    out = pl.pallas_call(
        _final_body,
        grid=(grid,),
        in_specs=[pl.BlockSpec((NC, NS, RB), lambda i: (0, 0, i)),
                  pl.BlockSpec((NC, RB, CP2), lambda i: (0, i, 0)),
                  pl.BlockSpec((RB, CP2), lambda i: (i, 0)),
                  pl.BlockSpec((1, CP2), lambda i: (0, 0))],
        out_specs=pl.BlockSpec((RB, CP2), lambda i: (i, 0)),
        out_shape=jax.ShapeDtypeStruct((NPAD, CP2), f32),
    )(deg, agg2, hp2, b2r)

    return out[:N, :CLS]


# speedup vs baseline: 7.7466x; 1.0120x over previous
import dataclasses
import functools

import jax
import jax.numpy as jnp
from jax import lax
from jax.experimental import pallas as pl
from jax.experimental.pallas import tpu as pltpu
from jax.experimental.pallas import tpu_sc as plsc

N = 10000
F = 256
CLS = 3
CP2 = 128
NC, NS = 2, 16
NW = NC * NS
L = 16
NPAD = 10240
E = 160000
EPAD = 163840
B = 64
BD = 128
FH = F // NC
STRIPE = NPAD // NS
RB = 512

_mesh = plsc.VectorSubcoreMesh(core_axis_name="c", subcore_axis_name="s")

_cp_no_layout = pltpu.CompilerParams()
if "needs_layout_passes" in pltpu.CompilerParams.__dataclass_fields__:
    _cp_no_layout = dataclasses.replace(_cp_no_layout, needs_layout_passes=False)


@functools.partial(
    pl.kernel,
    mesh=_mesh,
    out_type=jax.ShapeDtypeStruct((NC, NS, NPAD), jnp.float32),
    scratch_types=[
        pltpu.VMEM((BD,), jnp.int32),
        pltpu.VMEM((NPAD,), jnp.float32),
    ],
    compiler_params=_cp_no_layout,
)
def _deg_kernel(dst_hbm, zeros_hbm, out_hbm, dst_v, hist):
    c = lax.axis_index("c")
    s = lax.axis_index("s")
    pltpu.sync_copy(zeros_hbm, hist)
    ones16 = jnp.ones((L,), jnp.float32)
    per_w = EPAD // NW
    base = (c * NS + s) * per_w

    @pl.loop(0, per_w // BD)
    def _(k):
        pltpu.sync_copy(dst_hbm.at[pl.ds(base + k * BD, BD)], dst_v)

        @pl.loop(0, BD, step=L)
        def _(j):
            plsc.addupdate_scatter(hist, [dst_v[pl.ds(j, L)]], ones16)

    pltpu.sync_copy(hist, out_hbm.at[c].at[s])


R4 = 4


def _make_agg(feat_w, split_features):
    per_tile = EPAD // NS if split_features else EPAD // NC // NS
    K = per_tile // B
    R8 = 2 * R4

    @functools.partial(
        pl.kernel,
        mesh=_mesh,
        out_type=jax.ShapeDtypeStruct((NC, NPAD, feat_w), jnp.float32),
        scratch_types=(
            [pltpu.VMEM((R8, B), jnp.int32), pltpu.VMEM((R8, B), jnp.int32)]
            + [pltpu.VMEM((B, feat_w), jnp.float32) for _ in range(R4)]
            + [pltpu.VMEM_SHARED((NPAD, feat_w), jnp.float32)]
            + [pltpu.SemaphoreType.DMA for _ in range(2 * R4 + 2 * R8)]
        ),
    )
    def agg(hp_hbm, srcr_hbm, dstr_hbm, zeros_hbm, out_hbm,
            isrc, idst, *rest):
        rows = rest[:R4]
        accum = rest[R4]
        sems = rest[R4 + 1:]
        gsem = sems[:R4]
        ssem = sems[R4:2 * R4]
        isem = sems[2 * R4:2 * R4 + R8]
        dsem = sems[2 * R4 + R8:]
        c = lax.axis_index("c")
        s = lax.axis_index("s")
        hp = hp_hbm.at[c] if split_features else hp_hbm
        w = s if split_features else c * NS + s
        srcw = srcr_hbm.at[w]
        dstw = dstr_hbm.at[w]

        def load_idx(k, u):
            pltpu.async_copy(srcw.at[k], isrc.at[u], isem[u])
            pltpu.async_copy(dstw.at[k], idst.at[u], dsem[u])

        def wait_src(u):
            pltpu.make_async_copy(srcw.at[0], isrc.at[u], isem[u]).wait()

        def wait_dst(u):
            pltpu.make_async_copy(dstw.at[0], idst.at[u], dsem[u]).wait()

        def start_gather(u, j):
            pltpu.async_copy(hp.at[isrc.at[u]], rows[j], gsem[j])

        def wait_gather(j):
            pltpu.make_async_copy(hp.at[isrc.at[0]], rows[j], gsem[j]).wait()

        def start_scatter(j, u):
            pltpu.async_copy(rows[j], accum.at[idst.at[u]], ssem[j],
                             add=True)

        def wait_scatter(j):
            pltpu.make_async_copy(rows[j], accum.at[idst.at[0]],
                                  ssem[j]).wait()

        pltpu.sync_copy(zeros_hbm, accum.at[pl.ds(s * STRIPE, STRIPE)])
        for u in range(6):
            load_idx(u, u)
        wait_src(0)
        start_gather(0, 0)
        wait_src(1)
        start_gather(1, 1)
        plsc.subcore_barrier()

        def step(k, t8, first8, last8):
            t, u = t8 % R4, t8
            wait_gather(t)
            wait_dst(u)
            start_scatter(t, u)
            if not (first8 and t8 < 2):
                wait_scatter((t + 2) % R4)
            if not last8 or t8 < 2:
                load_idx(k + 6, (u + 6) % R8)
            if not last8 or t8 < 6:
                wait_src((u + 2) % R8)
                start_gather((u + 2) % R8, (t + 2) % R4)

        for t8 in range(R8):
            step(t8, t8, True, False)

        @pl.loop(1, K // R8 - 1)
        def _(q):
            for t8 in range(R8):
                step(q * R8 + t8, t8, False, False)

        for t8 in range(R8):
            step(t8 + K - R8, t8, False, True)
        for t in range(2):
            wait_scatter((K - 2 + t) % R4)

        plsc.subcore_barrier()
        pltpu.sync_copy(
            accum.at[pl.ds(s * STRIPE, STRIPE)],
            out_hbm.at[c].at[pl.ds(s * STRIPE, STRIPE)],
        )

    return agg


_agg1_kernel = _make_agg(FH, True)
_agg2_kernel = _make_agg(CP2, False)


def _mm1_body(x_ref, w_ref, o_ref):
    o_ref[...] = jnp.dot(x_ref[...], w_ref[...],
                         preferred_element_type=jnp.float32,
                         precision=lax.Precision.HIGHEST)


def _dis_of(deg_ref):
    deg = jnp.sum(deg_ref[...], axis=(0, 1)) + 1.0
    return lax.rsqrt(deg)[:, None]


def _scale_body(deg_ref, h_ref, o_ref):
    o_ref[0] = _dis_of(deg_ref) * h_ref[...]


def _layer2_body(deg_ref, a_ref, hp_ref, b1_ref, w2_ref, o_ref):
    dis = _dis_of(deg_ref)
    agg = jnp.concatenate([a_ref[0] + hp_ref[0], a_ref[1] + hp_ref[1]], axis=1)
    h1 = jnp.maximum(dis * agg + b1_ref[...], 0.0)
    o_ref[...] = dis * jnp.dot(h1, w2_ref[...],
                               preferred_element_type=jnp.float32,
                               precision=lax.Precision.HIGHEST)


def _final_body(deg_ref, a_ref, hp2_ref, b2_ref, o_ref):
    dis = _dis_of(deg_ref)
    o_ref[...] = dis * (a_ref[0] + a_ref[1] + hp2_ref[...]) + b2_ref[...]


def kernel(x, edge_index, W1, b1, W2, b2):
    f32 = jnp.float32
    src = edge_index[0].astype(jnp.int32)
    dst = edge_index[1].astype(jnp.int32)
    pad = jnp.full((EPAD - E,), N, jnp.int32)
    src = jnp.concatenate([src, pad])
    dst = jnp.concatenate([dst, pad])

    xp = jnp.pad(x, ((0, NPAD - N), (0, 0)))
    b1r = b1.reshape(1, F)
    w2p = jnp.pad(W2, ((0, 0), (0, CP2 - CLS)))
    b2r = jnp.pad(b2, (0, CP2 - CLS)).reshape(1, CP2)

    zeros_n = jnp.zeros((NPAD,), f32)
    zeros_fh = jnp.zeros((STRIPE, FH), f32)

    k1 = EPAD // NS // B
    k2 = EPAD // NC // NS // B
    src_r1 = src.reshape(NS, k1, B)
    dst_r1 = dst.reshape(NS, k1, B)
    src_r2 = src.reshape(NC * NS, k2, B)
    dst_r2 = dst.reshape(NC * NS, k2, B)

    deg = _deg_kernel(dst, zeros_n)

    grid = NPAD // RB
    hraw = pl.pallas_call(
        _mm1_body,
        grid=(grid,),
        in_specs=[pl.BlockSpec((RB, F), lambda i: (i, 0)),
                  pl.BlockSpec((F, F), lambda i: (0, 0))],
        out_specs=pl.BlockSpec((RB, F), lambda i: (i, 0)),
        out_shape=jax.ShapeDtypeStruct((NPAD, F), f32),
    )(xp, W1)

    hp = pl.pallas_call(
        _scale_body,
        grid=(NC, grid),
        in_specs=[pl.BlockSpec((NC, NS, RB), lambda c, i: (0, 0, i)),
                  pl.BlockSpec((RB, FH), lambda c, i: (i, c))],
        out_specs=pl.BlockSpec((1, RB, FH), lambda c, i: (c, i, 0)),
        out_shape=jax.ShapeDtypeStruct((NC, NPAD, FH), f32),
    )(deg, hraw)

    agg1 = _agg1_kernel(hp, src_r1, dst_r1, zeros_fh)

    hp2 = pl.pallas_call(
        _layer2_body,
        grid=(grid,),
        in_specs=[pl.BlockSpec((NC, NS, RB), lambda i: (0, 0, i)),
                  pl.BlockSpec((NC, RB, FH), lambda i: (0, i, 0)),
                  pl.BlockSpec((NC, RB, FH), lambda i: (0, i, 0)),
                  pl.BlockSpec((1, F), lambda i: (0, 0)),
                  pl.BlockSpec((F, CP2), lambda i: (0, 0))],
        out_specs=pl.BlockSpec((RB, CP2), lambda i: (i, 0)),
        out_shape=jax.ShapeDtypeStruct((NPAD, CP2), f32),
    )(deg, agg1, hp, b1r, w2p)

    agg2 = _agg2_kernel(hp2, src_r2, dst_r2, zeros_fh)

    out = pl.pallas_call(
        _final_body,
        grid=(grid,),
        in_specs=[pl.BlockSpec((NC, NS, RB), lambda i: (0, 0, i)),
                  pl.BlockSpec((NC, RB, CP2), lambda i: (0, i, 0)),
                  pl.BlockSpec((RB, CP2), lambda i: (i, 0)),
                  pl.BlockSpec((1, CP2), lambda i: (0, 0))],
        out_specs=pl.BlockSpec((RB, CP2), lambda i: (i, 0)),
        out_shape=jax.ShapeDtypeStruct((NPAD, CP2), f32),
    )(deg, agg2, hp2, b2r)

    return out[:N, :CLS]


# speedup vs baseline: 16.1366x; 2.0831x over previous
import dataclasses
import functools

import jax
import jax.numpy as jnp
from jax import lax
from jax.experimental import pallas as pl
from jax.experimental.pallas import tpu as pltpu
from jax.experimental.pallas import tpu_sc as plsc

N = 10000
F = 256
CLS = 3
CP2 = 128
NC, NS = 2, 16
NW = NC * NS
L = 16
NPAD = 10240
E = 160000
EPAD = 163840
B = 64
BD = 128
FH = F // NC
STRIPE = NPAD // NS
RB = 512

_mesh = plsc.VectorSubcoreMesh(core_axis_name="c", subcore_axis_name="s")

_cp_no_layout = pltpu.CompilerParams()
if "needs_layout_passes" in pltpu.CompilerParams.__dataclass_fields__:
    _cp_no_layout = dataclasses.replace(_cp_no_layout, needs_layout_passes=False)


@functools.partial(
    pl.kernel,
    mesh=_mesh,
    out_type=jax.ShapeDtypeStruct((NC, NS, NPAD), jnp.float32),
    scratch_types=[
        pltpu.VMEM((BD,), jnp.int32),
        pltpu.VMEM((NPAD,), jnp.float32),
    ],
    compiler_params=_cp_no_layout,
)
def _deg_kernel(dst_hbm, zeros_hbm, out_hbm, dst_v, hist):
    c = lax.axis_index("c")
    s = lax.axis_index("s")
    pltpu.sync_copy(zeros_hbm, hist)
    ones16 = jnp.ones((L,), jnp.float32)
    per_w = EPAD // NW
    base = (c * NS + s) * per_w

    @pl.loop(0, per_w // BD)
    def _(k):
        pltpu.sync_copy(dst_hbm.at[pl.ds(base + k * BD, BD)], dst_v)

        @pl.loop(0, BD, step=L)
        def _(j):
            plsc.addupdate_scatter(hist, [dst_v[pl.ds(j, L)]], ones16)

    pltpu.sync_copy(hist, out_hbm.at[c].at[s])


R4 = 4


def _make_agg(feat_w, split_features):
    per_tile = EPAD // NS if split_features else EPAD // NC // NS
    K = per_tile // B
    R8 = 2 * R4

    @functools.partial(
        pl.kernel,
        mesh=_mesh,
        out_type=jax.ShapeDtypeStruct((NC, NPAD, feat_w), jnp.float32),
        scratch_types=(
            [pltpu.VMEM((R8, B), jnp.int32), pltpu.VMEM((R8, B), jnp.int32)]
            + [pltpu.VMEM((B, feat_w), jnp.float32) for _ in range(R4)]
            + [pltpu.VMEM_SHARED((NPAD, feat_w), jnp.float32)]
            + [pltpu.SemaphoreType.DMA for _ in range(2 * R4 + 2 * R8)]
        ),
    )
    def agg(hp_hbm, srcr_hbm, dstr_hbm, zeros_hbm, out_hbm,
            isrc, idst, *rest):
        rows = rest[:R4]
        accum = rest[R4]
        sems = rest[R4 + 1:]
        gsem = sems[:R4]
        ssem = sems[R4:2 * R4]
        isem = sems[2 * R4:2 * R4 + R8]
        dsem = sems[2 * R4 + R8:]
        c = lax.axis_index("c")
        s = lax.axis_index("s")
        hp = hp_hbm.at[c] if split_features else hp_hbm
        w = s if split_features else c * NS + s
        srcw = srcr_hbm.at[w]
        dstw = dstr_hbm.at[w]

        def load_idx(k, u):
            pltpu.async_copy(srcw.at[k], isrc.at[u], isem[u])
            pltpu.async_copy(dstw.at[k], idst.at[u], dsem[u])

        def wait_src(u):
            pltpu.make_async_copy(srcw.at[0], isrc.at[u], isem[u]).wait()

        def wait_dst(u):
            pltpu.make_async_copy(dstw.at[0], idst.at[u], dsem[u]).wait()

        def start_gather(u, j):
            pltpu.async_copy(hp.at[isrc.at[u]], rows[j], gsem[j])

        def wait_gather(j):
            pltpu.make_async_copy(hp.at[isrc.at[0]], rows[j], gsem[j]).wait()

        def start_scatter(j, u):
            pltpu.async_copy(rows[j], accum.at[idst.at[u]], ssem[j],
                             add=True)

        def wait_scatter(j):
            pltpu.make_async_copy(rows[j], accum.at[idst.at[0]],
                                  ssem[j]).wait()

        pltpu.sync_copy(zeros_hbm, accum.at[pl.ds(s * STRIPE, STRIPE)])
        for u in range(6):
            load_idx(u, u)
        wait_src(0)
        start_gather(0, 0)
        wait_src(1)
        start_gather(1, 1)
        plsc.subcore_barrier()

        def step(k, t8, first8, last8):
            t, u = t8 % R4, t8
            wait_gather(t)
            wait_dst(u)
            start_scatter(t, u)
            if not (first8 and t8 < 2):
                wait_scatter((t + 2) % R4)
            if not last8 or t8 < 2:
                load_idx(k + 6, (u + 6) % R8)
            if not last8 or t8 < 6:
                wait_src((u + 2) % R8)
                start_gather((u + 2) % R8, (t + 2) % R4)

        for t8 in range(R8):
            step(t8, t8, True, False)

        @pl.loop(1, K // R8 - 1)
        def _(q):
            for t8 in range(R8):
                step(q * R8 + t8, t8, False, False)

        for t8 in range(R8):
            step(t8 + K - R8, t8, False, True)
        for t in range(2):
            wait_scatter((K - 2 + t) % R4)

        plsc.subcore_barrier()
        pltpu.sync_copy(
            accum.at[pl.ds(s * STRIPE, STRIPE)],
            out_hbm.at[c].at[pl.ds(s * STRIPE, STRIPE)],
        )

    return agg


_agg1_kernel = _make_agg(FH, True)
_agg2_kernel = _make_agg(CP2, False)


def _mm1_body(x_ref, w_ref, o_ref):
    o_ref[...] = jnp.dot(x_ref[...], w_ref[...],
                         preferred_element_type=jnp.float32,
                         precision=lax.Precision.HIGHEST)


def _dis_of(deg_ref):
    deg = jnp.sum(deg_ref[...], axis=(0, 1)) + 1.0
    return lax.rsqrt(deg)[:, None]


def _scale_body(deg_ref, h_ref, o_ref):
    o_ref[0] = _dis_of(deg_ref) * h_ref[...]


def _layer2_body(deg_ref, a_ref, hp_ref, b1_ref, w2_ref, o_ref):
    dis = _dis_of(deg_ref)
    agg = jnp.concatenate([a_ref[0] + hp_ref[0], a_ref[1] + hp_ref[1]], axis=1)
    h1 = jnp.maximum(dis * agg + b1_ref[...], 0.0)
    o_ref[...] = dis * jnp.dot(h1, w2_ref[...],
                               preferred_element_type=jnp.float32,
                               precision=lax.Precision.HIGHEST)


def _final_body(deg_ref, a_ref, hp2_ref, b2_ref, o_ref):
    dis = _dis_of(deg_ref)
    o_ref[...] = dis * (a_ref[0] + a_ref[1] + hp2_ref[...]) + b2_ref[...]


def kernel(x, edge_index, W1, b1, W2, b2):
    f32 = jnp.float32
    src = edge_index[0].astype(jnp.int32)
    dst = edge_index[1].astype(jnp.int32)
    pad = N + (jnp.arange(EPAD - E, dtype=jnp.int32) % (NPAD - N))
    src = jnp.concatenate([src, pad])
    dst = jnp.concatenate([dst, pad])

    xp = jnp.pad(x, ((0, NPAD - N), (0, 0)))
    b1r = b1.reshape(1, F)
    w2p = jnp.pad(W2, ((0, 0), (0, CP2 - CLS)))
    b2r = jnp.pad(b2, (0, CP2 - CLS)).reshape(1, CP2)

    zeros_n = jnp.zeros((NPAD,), f32)
    zeros_fh = jnp.zeros((STRIPE, FH), f32)

    k1 = EPAD // NS // B
    k2 = EPAD // NC // NS // B
    src_r1 = src.reshape(NS, k1, B)
    dst_r1 = dst.reshape(NS, k1, B)
    src_r2 = src.reshape(NC * NS, k2, B)
    dst_r2 = dst.reshape(NC * NS, k2, B)

    deg = _deg_kernel(dst, zeros_n)

    grid = NPAD // RB
    hraw = pl.pallas_call(
        _mm1_body,
        grid=(grid,),
        in_specs=[pl.BlockSpec((RB, F), lambda i: (i, 0)),
                  pl.BlockSpec((F, F), lambda i: (0, 0))],
        out_specs=pl.BlockSpec((RB, F), lambda i: (i, 0)),
        out_shape=jax.ShapeDtypeStruct((NPAD, F), f32),
    )(xp, W1)

    hp = pl.pallas_call(
        _scale_body,
        grid=(NC, grid),
        in_specs=[pl.BlockSpec((NC, NS, RB), lambda c, i: (0, 0, i)),
                  pl.BlockSpec((RB, FH), lambda c, i: (i, c))],
        out_specs=pl.BlockSpec((1, RB, FH), lambda c, i: (c, i, 0)),
        out_shape=jax.ShapeDtypeStruct((NC, NPAD, FH), f32),
    )(deg, hraw)

    agg1 = _agg1_kernel(hp, src_r1, dst_r1, zeros_fh)

    hp2 = pl.pallas_call(
        _layer2_body,
        grid=(grid,),
        in_specs=[pl.BlockSpec((NC, NS, RB), lambda i: (0, 0, i)),
                  pl.BlockSpec((NC, RB, FH), lambda i: (0, i, 0)),
                  pl.BlockSpec((NC, RB, FH), lambda i: (0, i, 0)),
                  pl.BlockSpec((1, F), lambda i: (0, 0)),
                  pl.BlockSpec((F, CP2), lambda i: (0, 0))],
        out_specs=pl.BlockSpec((RB, CP2), lambda i: (i, 0)),
        out_shape=jax.ShapeDtypeStruct((NPAD, CP2), f32),
    )(deg, agg1, hp, b1r, w2p)

    agg2 = _agg2_kernel(hp2, src_r2, dst_r2, zeros_fh)

    out = pl.pallas_call(
        _final_body,
        grid=(grid,),
        in_specs=[pl.BlockSpec((NC, NS, RB), lambda i: (0, 0, i)),
                  pl.BlockSpec((NC, RB, CP2), lambda i: (0, i, 0)),
                  pl.BlockSpec((RB, CP2), lambda i: (i, 0)),
                  pl.BlockSpec((1, CP2), lambda i: (0, 0))],
        out_specs=pl.BlockSpec((RB, CP2), lambda i: (i, 0)),
        out_shape=jax.ShapeDtypeStruct((NPAD, CP2), f32),
    )(deg, agg2, hp2, b2r)

    return out[:N, :CLS]


# speedup vs baseline: 16.1385x; 1.0001x over previous
import dataclasses
import functools

import jax
import jax.numpy as jnp
from jax import lax
from jax.experimental import pallas as pl
from jax.experimental.pallas import tpu as pltpu
from jax.experimental.pallas import tpu_sc as plsc

N = 10000
F = 256
CLS = 3
CP = 16
CP2 = 128
NC, NS = 2, 16
NW = NC * NS
L = 16
NPAD = 10240
E = 160000
EPAD = 163840
B = 64
BD = 1024
FH = F // NC
STRIPE = NPAD // NS
RB = 512

_mesh = plsc.VectorSubcoreMesh(core_axis_name="c", subcore_axis_name="s")

_cp_no_layout = pltpu.CompilerParams()
if "needs_layout_passes" in pltpu.CompilerParams.__dataclass_fields__:
    _cp_no_layout = dataclasses.replace(_cp_no_layout, needs_layout_passes=False)


@functools.partial(
    pl.kernel,
    mesh=_mesh,
    out_type=jax.ShapeDtypeStruct((NC, NS, NPAD), jnp.float32),
    scratch_types=[
        pltpu.VMEM((BD,), jnp.int32),
        pltpu.VMEM((NPAD,), jnp.float32),
    ],
    compiler_params=_cp_no_layout,
)
def _deg_kernel(dst_hbm, zeros_hbm, out_hbm, dst_v, hist):
    c = lax.axis_index("c")
    s = lax.axis_index("s")
    pltpu.sync_copy(zeros_hbm, hist)
    ones16 = jnp.ones((L,), jnp.float32)
    per_w = EPAD // NW
    base = (c * NS + s) * per_w

    @pl.loop(0, per_w // BD)
    def _(k):
        pltpu.sync_copy(dst_hbm.at[pl.ds(base + k * BD, BD)], dst_v)

        @pl.loop(0, BD, step=L)
        def _(j):
            plsc.addupdate_scatter(hist, [dst_v[pl.ds(j, L)]], ones16)

    pltpu.sync_copy(hist, out_hbm.at[c].at[s])


R4 = 4


def _make_agg(feat_w, split_features, out_w=None):
    out_w = feat_w if out_w is None else out_w
    per_tile = EPAD // NS if split_features else EPAD // NC // NS
    K = per_tile // B
    R8 = 2 * R4

    @functools.partial(
        pl.kernel,
        mesh=_mesh,
        out_type=jax.ShapeDtypeStruct((NC, NPAD, out_w), jnp.float32),
        scratch_types=(
            [pltpu.VMEM((R8, B), jnp.int32), pltpu.VMEM((R8, B), jnp.int32)]
            + [pltpu.VMEM((B, feat_w), jnp.float32) for _ in range(R4)]
            + [pltpu.VMEM_SHARED((NPAD, feat_w), jnp.float32)]
            + [pltpu.SemaphoreType.DMA for _ in range(2 * R4 + 2 * R8)]
        ),
    )
    def agg(hp_hbm, srcr_hbm, dstr_hbm, zeros_hbm, out_hbm,
            isrc, idst, *rest):
        rows = rest[:R4]
        accum = rest[R4]
        sems = rest[R4 + 1:]
        gsem = sems[:R4]
        ssem = sems[R4:2 * R4]
        isem = sems[2 * R4:2 * R4 + R8]
        dsem = sems[2 * R4 + R8:]
        c = lax.axis_index("c")
        s = lax.axis_index("s")
        hp = hp_hbm.at[c] if split_features else hp_hbm
        w = s if split_features else c * NS + s
        srcw = srcr_hbm.at[w]
        dstw = dstr_hbm.at[w]

        def load_idx(k, u):
            pltpu.async_copy(srcw.at[k], isrc.at[u], isem[u])
            pltpu.async_copy(dstw.at[k], idst.at[u], dsem[u])

        def wait_src(u):
            pltpu.make_async_copy(srcw.at[0], isrc.at[u], isem[u]).wait()

        def wait_dst(u):
            pltpu.make_async_copy(dstw.at[0], idst.at[u], dsem[u]).wait()

        def start_gather(u, j):
            pltpu.async_copy(hp.at[isrc.at[u]], rows[j], gsem[j])

        def wait_gather(j):
            pltpu.make_async_copy(hp.at[isrc.at[0]], rows[j], gsem[j]).wait()

        def start_scatter(j, u):
            pltpu.async_copy(rows[j], accum.at[idst.at[u]], ssem[j],
                             add=True)

        def wait_scatter(j):
            pltpu.make_async_copy(rows[j], accum.at[idst.at[0]],
                                  ssem[j]).wait()

        pltpu.sync_copy(zeros_hbm,
                        accum.at[pl.ds(s * STRIPE, STRIPE), pl.ds(0, out_w)])
        for u in range(6):
            load_idx(u, u)
        wait_src(0)
        start_gather(0, 0)
        wait_src(1)
        start_gather(1, 1)
        plsc.subcore_barrier()

        def step(k, t8, first8, last8):
            t, u = t8 % R4, t8
            wait_gather(t)
            wait_dst(u)
            start_scatter(t, u)
            if not (first8 and t8 < 2):
                wait_scatter((t + 2) % R4)
            if not last8 or t8 < 2:
                load_idx(k + 6, (u + 6) % R8)
            if not last8 or t8 < 6:
                wait_src((u + 2) % R8)
                start_gather((u + 2) % R8, (t + 2) % R4)

        for t8 in range(R8):
            step(t8, t8, True, False)

        @pl.loop(1, K // R8 - 1)
        def _(q):
            for t8 in range(R8):
                step(q * R8 + t8, t8, False, False)

        for t8 in range(R8):
            step(t8 + K - R8, t8, False, True)
        for t in range(2):
            wait_scatter((K - 2 + t) % R4)

        plsc.subcore_barrier()
        pltpu.sync_copy(
            accum.at[pl.ds(s * STRIPE, STRIPE), pl.ds(0, out_w)],
            out_hbm.at[c].at[pl.ds(s * STRIPE, STRIPE)],
        )

    return agg


_agg1_kernel = _make_agg(FH, True)
_agg2_kernel = _make_agg(CP2, False)


def _mm1_body(x_ref, w_ref, o_ref):
    o_ref[...] = jnp.dot(x_ref[...], w_ref[...],
                         preferred_element_type=jnp.float32,
                         precision=lax.Precision.HIGHEST)


def _dis_of(deg_ref):
    deg = jnp.sum(deg_ref[...], axis=(0, 1)) + 1.0
    return lax.rsqrt(deg)[:, None]


def _scale_body(deg_ref, h_ref, o_ref):
    o_ref[0] = _dis_of(deg_ref) * h_ref[...]


def _layer2_body(deg_ref, a_ref, hp_ref, b1_ref, w2_ref, o_ref):
    dis = _dis_of(deg_ref)
    agg = jnp.concatenate([a_ref[0] + hp_ref[0], a_ref[1] + hp_ref[1]], axis=1)
    h1 = jnp.maximum(dis * agg + b1_ref[...], 0.0)
    o_ref[...] = dis * jnp.dot(h1, w2_ref[...],
                               preferred_element_type=jnp.float32,
                               precision=lax.Precision.HIGHEST)


def _final_body(deg_ref, a_ref, hp2_ref, b2_ref, o_ref):
    dis = _dis_of(deg_ref)
    o_ref[...] = dis * (a_ref[0] + a_ref[1] + hp2_ref[...]) + b2_ref[...]


def kernel(x, edge_index, W1, b1, W2, b2):
    f32 = jnp.float32
    src = edge_index[0].astype(jnp.int32)
    dst = edge_index[1].astype(jnp.int32)
    pad = N + (jnp.arange(EPAD - E, dtype=jnp.int32) % (NPAD - N))
    src = jnp.concatenate([src, pad])
    dst = jnp.concatenate([dst, pad])

    xp = jnp.pad(x, ((0, NPAD - N), (0, 0)))
    b1r = b1.reshape(1, F)
    w2p = jnp.pad(W2, ((0, 0), (0, CP2 - CLS)))
    b2r = jnp.pad(b2, (0, CP2 - CLS)).reshape(1, CP2)

    zeros_n = jnp.zeros((NPAD,), f32)
    zeros_fh = jnp.zeros((STRIPE, FH), f32)

    k1 = EPAD // NS // B
    k2 = EPAD // NC // NS // B
    src_r1 = src.reshape(NS, k1, B)
    dst_r1 = dst.reshape(NS, k1, B)
    src_r2 = src.reshape(NC * NS, k2, B)
    dst_r2 = dst.reshape(NC * NS, k2, B)

    deg = _deg_kernel(dst, zeros_n)

    grid = NPAD // RB
    hraw = pl.pallas_call(
        _mm1_body,
        grid=(grid,),
        in_specs=[pl.BlockSpec((RB, F), lambda i: (i, 0)),
                  pl.BlockSpec((F, F), lambda i: (0, 0))],
        out_specs=pl.BlockSpec((RB, F), lambda i: (i, 0)),
        out_shape=jax.ShapeDtypeStruct((NPAD, F), f32),
    )(xp, W1)

    hp = pl.pallas_call(
        _scale_body,
        grid=(NC, grid),
        in_specs=[pl.BlockSpec((NC, NS, RB), lambda c, i: (0, 0, i)),
                  pl.BlockSpec((RB, FH), lambda c, i: (i, c))],
        out_specs=pl.BlockSpec((1, RB, FH), lambda c, i: (c, i, 0)),
        out_shape=jax.ShapeDtypeStruct((NC, NPAD, FH), f32),
    )(deg, hraw)

    agg1 = _agg1_kernel(hp, src_r1, dst_r1, zeros_fh)

    hp2 = pl.pallas_call(
        _layer2_body,
        grid=(grid,),
        in_specs=[pl.BlockSpec((NC, NS, RB), lambda i: (0, 0, i)),
                  pl.BlockSpec((NC, RB, FH), lambda i: (0, i, 0)),
                  pl.BlockSpec((NC, RB, FH), lambda i: (0, i, 0)),
                  pl.BlockSpec((1, F), lambda i: (0, 0)),
                  pl.BlockSpec((F, CP2), lambda i: (0, 0))],
        out_specs=pl.BlockSpec((RB, CP2), lambda i: (i, 0)),
        out_shape=jax.ShapeDtypeStruct((NPAD, CP2), f32),
    )(deg, agg1, hp, b1r, w2p)

    agg2 = _agg2_kernel(hp2, src_r2, dst_r2, zeros_fh)

    out = pl.pallas_call(
        _final_body,
        grid=(grid,),
        in_specs=[pl.BlockSpec((NC, NS, RB), lambda i: (0, 0, i)),
                  pl.BlockSpec((NC, RB, CP2), lambda i: (0, i, 0)),
                  pl.BlockSpec((RB, CP2), lambda i: (i, 0)),
                  pl.BlockSpec((1, CP2), lambda i: (0, 0))],
        out_specs=pl.BlockSpec((RB, CP2), lambda i: (i, 0)),
        out_shape=jax.ShapeDtypeStruct((NPAD, CP2), f32),
    )(deg, agg2, hp2, b2r)

    return out[:N, :CLS]


# speedup vs baseline: 17.1317x; 1.0615x over previous
import dataclasses
import functools

import jax
import jax.numpy as jnp
from jax import lax
from jax.experimental import pallas as pl
from jax.experimental.pallas import tpu as pltpu
from jax.experimental.pallas import tpu_sc as plsc

N = 10000
F = 256
CLS = 3
CP = 16
CP2 = 128
NC, NS = 2, 16
NW = NC * NS
L = 16
NPAD = 10240
E = 160000
EPAD = 163840
B = 64
BD = 1024
FH = F // NC
STRIPE = NPAD // NS
RB = 512

_mesh = plsc.VectorSubcoreMesh(core_axis_name="c", subcore_axis_name="s")

_cp_no_layout = pltpu.CompilerParams()
if "needs_layout_passes" in pltpu.CompilerParams.__dataclass_fields__:
    _cp_no_layout = dataclasses.replace(_cp_no_layout, needs_layout_passes=False)


@functools.partial(
    pl.kernel,
    mesh=_mesh,
    out_type=jax.ShapeDtypeStruct((NC, NS, NPAD), jnp.float32),
    scratch_types=[
        pltpu.VMEM((BD,), jnp.int32),
        pltpu.VMEM((NPAD,), jnp.float32),
    ],
    compiler_params=_cp_no_layout,
)
def _deg_kernel(dst_hbm, zeros_hbm, out_hbm, dst_v, hist):
    c = lax.axis_index("c")
    s = lax.axis_index("s")
    pltpu.sync_copy(zeros_hbm, hist)
    ones16 = jnp.ones((L,), jnp.float32)
    per_w = EPAD // NW
    base = (c * NS + s) * per_w

    @pl.loop(0, per_w // BD)
    def _(k):
        pltpu.sync_copy(dst_hbm.at[pl.ds(base + k * BD, BD)], dst_v)

        @pl.loop(0, BD, step=L)
        def _(j):
            plsc.addupdate_scatter(hist, [dst_v[pl.ds(j, L)]], ones16)

    pltpu.sync_copy(hist, out_hbm.at[c].at[s])


R4 = 4


def _make_agg(feat_w, split_features, out_w=None):
    out_w = feat_w if out_w is None else out_w
    per_tile = EPAD // NS if split_features else EPAD // NC // NS
    K = per_tile // B
    R8 = 2 * R4

    @functools.partial(
        pl.kernel,
        mesh=_mesh,
        out_type=jax.ShapeDtypeStruct((NC, NPAD, out_w), jnp.float32),
        scratch_types=(
            [pltpu.VMEM((R8, B), jnp.int32), pltpu.VMEM((R8, B), jnp.int32)]
            + [pltpu.VMEM((B, feat_w), jnp.float32) for _ in range(R4)]
            + [pltpu.VMEM_SHARED((NPAD, feat_w), jnp.float32)]
            + [pltpu.SemaphoreType.DMA for _ in range(2 * R4 + 2 * R8)]
        ),
    )
    def agg(hp_hbm, srcr_hbm, dstr_hbm, zeros_hbm, out_hbm,
            isrc, idst, *rest):
        rows = rest[:R4]
        accum = rest[R4]
        sems = rest[R4 + 1:]
        gsem = sems[:R4]
        ssem = sems[R4:2 * R4]
        isem = sems[2 * R4:2 * R4 + R8]
        dsem = sems[2 * R4 + R8:]
        c = lax.axis_index("c")
        s = lax.axis_index("s")
        hp = hp_hbm.at[c] if split_features else hp_hbm
        w = s if split_features else c * NS + s
        srcw = srcr_hbm.at[w]
        dstw = dstr_hbm.at[w]

        def load_idx(k, u):
            pltpu.async_copy(srcw.at[k], isrc.at[u], isem[u])
            pltpu.async_copy(dstw.at[k], idst.at[u], dsem[u])

        def wait_src(u):
            pltpu.make_async_copy(srcw.at[0], isrc.at[u], isem[u]).wait()

        def wait_dst(u):
            pltpu.make_async_copy(dstw.at[0], idst.at[u], dsem[u]).wait()

        def start_gather(u, j):
            pltpu.async_copy(hp.at[isrc.at[u]], rows[j], gsem[j])

        def wait_gather(j):
            pltpu.make_async_copy(hp.at[isrc.at[0]], rows[j], gsem[j]).wait()

        def start_scatter(j, u):
            pltpu.async_copy(rows[j], accum.at[idst.at[u]], ssem[j],
                             add=True)

        def wait_scatter(j):
            pltpu.make_async_copy(rows[j], accum.at[idst.at[0]],
                                  ssem[j]).wait()

        pltpu.sync_copy(zeros_hbm,
                        accum.at[pl.ds(s * STRIPE, STRIPE), pl.ds(0, out_w)])
        for u in range(6):
            load_idx(u, u)
        wait_src(0)
        start_gather(0, 0)
        wait_src(1)
        start_gather(1, 1)
        plsc.subcore_barrier()

        def step(k, t8, first8, last8):
            t, u = t8 % R4, t8
            wait_gather(t)
            wait_dst(u)
            start_scatter(t, u)
            if not (first8 and t8 < 2):
                wait_scatter((t + 2) % R4)
            if not last8 or t8 < 2:
                load_idx(k + 6, (u + 6) % R8)
            if not last8 or t8 < 6:
                wait_src((u + 2) % R8)
                start_gather((u + 2) % R8, (t + 2) % R4)

        for t8 in range(R8):
            step(t8, t8, True, False)

        @pl.loop(1, K // R8 - 1)
        def _(q):
            for t8 in range(R8):
                step(q * R8 + t8, t8, False, False)

        for t8 in range(R8):
            step(t8 + K - R8, t8, False, True)
        for t in range(2):
            wait_scatter((K - 2 + t) % R4)

        plsc.subcore_barrier()
        pltpu.sync_copy(
            accum.at[pl.ds(s * STRIPE, STRIPE), pl.ds(0, out_w)],
            out_hbm.at[c].at[pl.ds(s * STRIPE, STRIPE)],
        )

    return agg


_agg1_kernel = _make_agg(FH, True)
_agg2_kernel = _make_agg(CP2, False)


def _mm1s_body(deg_ref, x_ref, w_ref, o_ref):
    xs = _dis_of(deg_ref) * x_ref[...]
    o_ref[0] = jnp.dot(xs, w_ref[...], preferred_element_type=jnp.float32)


def _dis_of(deg_ref):
    deg = jnp.sum(deg_ref[...], axis=(0, 1)) + 1.0
    return lax.rsqrt(deg)[:, None]


def _layer2_body(deg_ref, a_ref, hp_ref, b1_ref, w2_ref, o_ref):
    dis = _dis_of(deg_ref)
    agg = jnp.concatenate([a_ref[0] + hp_ref[0], a_ref[1] + hp_ref[1]], axis=1)
    h1 = jnp.maximum(dis * agg + b1_ref[...], 0.0)
    o_ref[...] = dis * jnp.dot(h1, w2_ref[...],
                               preferred_element_type=jnp.float32)


def _final_body(deg_ref, a_ref, hp2_ref, b2_ref, o_ref):
    dis = _dis_of(deg_ref)
    o_ref[...] = dis * (a_ref[0] + a_ref[1] + hp2_ref[...]) + b2_ref[...]


def kernel(x, edge_index, W1, b1, W2, b2):
    f32 = jnp.float32
    src = edge_index[0].astype(jnp.int32)
    dst = edge_index[1].astype(jnp.int32)
    pad = N + (jnp.arange(EPAD - E, dtype=jnp.int32) % (NPAD - N))
    src = jnp.concatenate([src, pad])
    dst = jnp.concatenate([dst, pad])

    xp = jnp.pad(x, ((0, NPAD - N), (0, 0)))
    b1r = b1.reshape(1, F)
    w2p = jnp.pad(W2, ((0, 0), (0, CP2 - CLS)))
    b2r = jnp.pad(b2, (0, CP2 - CLS)).reshape(1, CP2)

    zeros_n = jnp.zeros((NPAD,), f32)
    zeros_fh = jnp.zeros((STRIPE, FH), f32)

    k1 = EPAD // NS // B
    k2 = EPAD // NC // NS // B
    src_r1 = src.reshape(NS, k1, B)
    dst_r1 = dst.reshape(NS, k1, B)
    src_r2 = src.reshape(NC * NS, k2, B)
    dst_r2 = dst.reshape(NC * NS, k2, B)

    deg = _deg_kernel(dst, zeros_n)

    grid = NPAD // RB
    hp = pl.pallas_call(
        _mm1s_body,
        grid=(NC, grid),
        in_specs=[pl.BlockSpec((NC, NS, RB), lambda c, i: (0, 0, i)),
                  pl.BlockSpec((RB, F), lambda c, i: (i, 0)),
                  pl.BlockSpec((F, FH), lambda c, i: (0, c))],
        out_specs=pl.BlockSpec((1, RB, FH), lambda c, i: (c, i, 0)),
        out_shape=jax.ShapeDtypeStruct((NC, NPAD, FH), f32),
    )(deg, xp, W1)

    agg1 = _agg1_kernel(hp, src_r1, dst_r1, zeros_fh)

    hp2 = pl.pallas_call(
        _layer2_body,
        grid=(grid,),
        in_specs=[pl.BlockSpec((NC, NS, RB), lambda i: (0, 0, i)),
                  pl.BlockSpec((NC, RB, FH), lambda i: (0, i, 0)),
                  pl.BlockSpec((NC, RB, FH), lambda i: (0, i, 0)),
                  pl.BlockSpec((1, F), lambda i: (0, 0)),
                  pl.BlockSpec((F, CP2), lambda i: (0, 0))],
        out_specs=pl.BlockSpec((RB, CP2), lambda i: (i, 0)),
        out_shape=jax.ShapeDtypeStruct((NPAD, CP2), f32),
    )(deg, agg1, hp, b1r, w2p)

    agg2 = _agg2_kernel(hp2, src_r2, dst_r2, zeros_fh)

    out = pl.pallas_call(
        _final_body,
        grid=(grid,),
        in_specs=[pl.BlockSpec((NC, NS, RB), lambda i: (0, 0, i)),
                  pl.BlockSpec((NC, RB, CP2), lambda i: (0, i, 0)),
                  pl.BlockSpec((RB, CP2), lambda i: (i, 0)),
                  pl.BlockSpec((1, CP2), lambda i: (0, 0))],
        out_specs=pl.BlockSpec((RB, CP2), lambda i: (i, 0)),
        out_shape=jax.ShapeDtypeStruct((NPAD, CP2), f32),
    )(deg, agg2, hp2, b2r)

    return out[:N, :CLS]
